# baseline (speedup 1.0000x reference)
import functools

import jax
import jax.numpy as jnp
from jax import lax
from jax.experimental import pallas as pl
from jax.experimental.pallas import tpu as pltpu
from jax.experimental.pallas import tpu_sc as plsc

B = 16384
D = 16
NC = 2
NS = 16
NW = NC * NS
RPW = B // NW
L = 16


def _body(uidx_hbm, iidx_hbm, ue_hbm, ie_hbm, ub_hbm, ib_hbm, out_hbm,
          uidx_v, iidx_v, ue_v, ie_v, ub_v, ib_v, out_v,
          sem0, sem1, sem2, sem3):
    wid = lax.axis_index("s") * NC + lax.axis_index("c")
    base = wid * RPW

    pltpu.sync_copy(uidx_hbm.at[pl.ds(base, RPW)], uidx_v)
    pltpu.sync_copy(iidx_hbm.at[pl.ds(base, RPW)], iidx_v)

    cp0 = pltpu.async_copy(ue_hbm.at[uidx_v], ue_v, sem0)
    cp1 = pltpu.async_copy(ie_hbm.at[iidx_v], ie_v, sem1)
    cp2 = pltpu.async_copy(ub_hbm.at[uidx_v], ub_v, sem2)
    cp3 = pltpu.async_copy(ib_hbm.at[iidx_v], ib_v, sem3)
    cp0.wait()
    cp1.wait()
    cp2.wait()
    cp3.wait()

    inv_d = jnp.float32(1.0 / D)
    lanes = lax.iota(jnp.int32, L)

    def grp_body(j, carry):
        off = j * L
        acc = ub_v[pl.ds(off, L)] + ib_v[pl.ds(off, L)]
        for r in range(L):
            u = ue_v[off + r, :]
            i = ie_v[off + r, :]
            s = jnp.sum(u * (i + inv_d), axis=0)
            acc = jnp.where(lanes == r, s, acc)
        out_v[pl.ds(off, L)] = 1.0 / (1.0 + jnp.exp(-acc))
        return carry

    lax.fori_loop(0, RPW // L, grp_body, 0, unroll=2)

    pltpu.sync_copy(out_v, out_hbm.at[pl.ds(base, RPW)])


@jax.jit
def _svdpp(uidx, iidx, user_emb, item_emb, ub, ib):
    mesh = plsc.VectorSubcoreMesh(
        core_axis_name="c", subcore_axis_name="s", num_cores=NC)
    f = functools.partial(
        pl.kernel,
        out_type=jax.ShapeDtypeStruct((B,), jnp.float32),
        mesh=mesh,
        scratch_types=[
            pltpu.VMEM((RPW,), jnp.int32),
            pltpu.VMEM((RPW,), jnp.int32),
            pltpu.VMEM((RPW, D), jnp.float32),
            pltpu.VMEM((RPW, D), jnp.float32),
            pltpu.VMEM((RPW,), jnp.float32),
            pltpu.VMEM((RPW,), jnp.float32),
            pltpu.VMEM((RPW,), jnp.float32),
            pltpu.SemaphoreType.DMA,
            pltpu.SemaphoreType.DMA,
            pltpu.SemaphoreType.DMA,
            pltpu.SemaphoreType.DMA,
        ],
        compiler_params=pltpu.CompilerParams(
            needs_layout_passes=False, use_tc_tiling_on_sc=False),
    )(_body)
    return f(uidx, iidx, user_emb, item_emb, ub, ib)


def kernel(x, user_emb, item_emb, user_bias, item_bias):
    uidx = x[:, 0]
    iidx = x[:, 1]
    ub = jnp.reshape(user_bias, (-1,))
    ib = jnp.reshape(item_bias, (-1,))
    return _svdpp(uidx, iidx, user_emb, item_emb, ub, ib)


# speedup vs baseline: 4.0814x; 4.0814x over previous
import functools

import jax
import jax.numpy as jnp
from jax import lax
from jax.experimental import pallas as pl
from jax.experimental.pallas import tpu as pltpu
from jax.experimental.pallas import tpu_sc as plsc

B = 16384
D = 16
NC = 2
NS = 16
NW = NC * NS
RPW = B // NW
L = 16
NG = RPW // L


def _fire_group(tbl_hbm, idx_v, chunk_v, sem, j, jb):
    g = idx_v[pl.ds(j * L, L)]
    cvec = (g >> 7) << 7
    for r in range(L):
        c = pl.multiple_of(cvec[r], 128)
        pltpu.async_copy(
            tbl_hbm.at[:, pl.ds(c, 128)], chunk_v.at[jb, r], sem)


def _drain_group(tbl_hbm, chunk_v, sem):
    for _ in range(L):
        pltpu.make_async_copy(
            tbl_hbm.at[:, pl.ds(0, 128)], chunk_v.at[0, 0], sem).wait()


def _body(uidx_hbm, iidx_hbm, uet_hbm, iet_hbm, ub_hbm, ib_hbm, out_hbm,
          uidx_v, iidx_v, chunk_v, ut_v, ub_v, ib_v, out_v,
          sem_d, sem_b):
    wid = lax.axis_index("s") * NC + lax.axis_index("c")
    base = wid * RPW

    pltpu.sync_copy(uidx_hbm.at[pl.ds(base, RPW)], uidx_v)
    pltpu.sync_copy(iidx_hbm.at[pl.ds(base, RPW)], iidx_v)

    cpb0 = pltpu.async_copy(ub_hbm.at[uidx_v], ub_v, sem_b)
    cpb1 = pltpu.async_copy(ib_hbm.at[iidx_v], ib_v, sem_b)

    lanes = lax.iota(jnp.int32, L)
    inv_d = jnp.float32(1.0 / D)

    _fire_group(uet_hbm, uidx_v, chunk_v, sem_d, 0, 0)

    def u_group(j, carry):
        jb = j & 1

        @pl.when(j < NG - 1)
        def _():
            _fire_group(uet_hbm, uidx_v, chunk_v, sem_d, j + 1, (j + 1) & 1)

        _drain_group(uet_hbm, chunk_v, sem_d)
        g = uidx_v[pl.ds(j * L, L)]
        lvec = g & 127
        jbv = jnp.full((L,), jb, jnp.int32)
        off = j * L
        for d in range(D):
            dvec = jnp.full((L,), d, jnp.int32)
            ut_v[d, pl.ds(off, L)] = plsc.load_gather(
                chunk_v, [jbv, lanes, dvec, lvec])
        return carry

    lax.fori_loop(0, NG, u_group, 0)

    cpb0.wait()
    cpb1.wait()

    _fire_group(iet_hbm, iidx_v, chunk_v, sem_d, 0, 0)

    def i_group(j, carry):
        jb = j & 1

        @pl.when(j < NG - 1)
        def _():
            _fire_group(iet_hbm, iidx_v, chunk_v, sem_d, j + 1, (j + 1) & 1)

        _drain_group(iet_hbm, chunk_v, sem_d)
        g = iidx_v[pl.ds(j * L, L)]
        lvec = g & 127
        jbv = jnp.full((L,), jb, jnp.int32)
        off = j * L
        acc = ub_v[pl.ds(off, L)] + ib_v[pl.ds(off, L)]
        for d in range(D):
            dvec = jnp.full((L,), d, jnp.int32)
            ivec = plsc.load_gather(chunk_v, [jbv, lanes, dvec, lvec])
            acc = acc + ut_v[d, pl.ds(off, L)] * (ivec + inv_d)
        out_v[pl.ds(off, L)] = 1.0 / (1.0 + jnp.exp(-acc))
        return carry

    lax.fori_loop(0, NG, i_group, 0)

    pltpu.sync_copy(out_v, out_hbm.at[pl.ds(base, RPW)])


@jax.jit
def _svdpp(uidx, iidx, uet, iet, ub, ib):
    mesh = plsc.VectorSubcoreMesh(
        core_axis_name="c", subcore_axis_name="s", num_cores=NC)
    f = functools.partial(
        pl.kernel,
        out_type=jax.ShapeDtypeStruct((B,), jnp.float32),
        mesh=mesh,
        scratch_types=[
            pltpu.VMEM((RPW,), jnp.int32),
            pltpu.VMEM((RPW,), jnp.int32),
            pltpu.VMEM((2, L, D, 128), jnp.float32),
            pltpu.VMEM((D, RPW), jnp.float32),
            pltpu.VMEM((RPW,), jnp.float32),
            pltpu.VMEM((RPW,), jnp.float32),
            pltpu.VMEM((RPW,), jnp.float32),
            pltpu.SemaphoreType.DMA,
            pltpu.SemaphoreType.DMA,
        ],
        compiler_params=pltpu.CompilerParams(needs_layout_passes=False),
    )(_body)
    return f(uidx, iidx, uet, iet, ub, ib)


def kernel(x, user_emb, item_emb, user_bias, item_bias):
    uidx = x[:, 0]
    iidx = x[:, 1]
    uet = user_emb.T
    iet = item_emb.T
    ub = jnp.reshape(user_bias, (-1,))
    ib = jnp.reshape(item_bias, (-1,))
    return _svdpp(uidx, iidx, uet, iet, ub, ib)
